# Initial kernel scaffold; baseline (speedup 1.0000x reference)
import jax
import jax.numpy as jnp
from jax.experimental import pallas as pl


def kernel(char_codes, lookup_table):
    raise NotImplementedError("write your pallas kernel here")


# speedup vs baseline: 544.0234x; 544.0234x over previous
import jax
import jax.numpy as jnp
from jax.experimental import pallas as pl

_BR = 512

_START_VAL = 60.0
_END_VAL = 61.0


def _tc_body(codes_ref, table_ref, out_ref):
    codes = codes_ref[...]
    table = table_ref[...]
    br = codes.shape[0]
    tb = jnp.broadcast_to(table, (br, 128))
    vals = jnp.take_along_axis(tb, codes, axis=1, mode="promise_in_bounds")
    start = jnp.full((br, 1), _START_VAL, jnp.float32)
    end = jnp.full((br, 1), _END_VAL, jnp.float32)
    out_ref[...] = jnp.concatenate([start, vals, end], axis=1)


def kernel(char_codes, lookup_table):
    B, L = char_codes.shape
    table2d = lookup_table.reshape(1, 128)
    return pl.pallas_call(
        _tc_body,
        grid=(B // _BR,),
        in_specs=[
            pl.BlockSpec((_BR, L), lambda i: (i, 0)),
            pl.BlockSpec((1, 128), lambda i: (0, 0)),
        ],
        out_specs=pl.BlockSpec((_BR, L + 2), lambda i: (i, 0)),
        out_shape=jax.ShapeDtypeStruct((B, L + 2), jnp.float32),
    )(char_codes, table2d)
